# baseline (speedup 1.0000x reference)
import jax
import jax.numpy as jnp
from jax.experimental import pallas as pl
from jax.experimental.pallas import tpu as pltpu

T, B, N = 4, 8, 512
BN = B * N
IN_DIM, HID, OUT = 128, 128, 64

BI = 512
NI = BN // BI


def _fused_kernel(m_ref, a_ref, x_ref, w1_ref, b1_ref, w2_ref, b2_ref,
                  wfc_ref, bfc_ref, out_ref, a8v, acc, g, deg, dis):
    j = pl.program_id(1)
    phase = j // NI
    i = j % NI

    @pl.when(phase == 0)
    def _():
        a = a_ref[0]
        a8v[pl.ds(i * BI, BI), :] = a.astype(jnp.bfloat16)
        mi = m_ref[0, 0, pl.ds(i * BI, BI)]
        part = jnp.dot(mi[None, :], a, preferred_element_type=jnp.float32)

        @pl.when(i == 0)
        def _():
            deg[...] = part

        @pl.when(i > 0)
        def _():
            deg[...] += part

    @pl.when(phase == 1)
    def _():
        @pl.when(i == 0)
        def _():
            m = m_ref[0, 0]
            d = deg[0] * m + m
            dis[...] = jnp.where(d > 0, jax.lax.rsqrt(d), 0.0)[None]
            h = jax.lax.dot_general(x_ref[0], w1_ref[...],
                                    (((1,), (1,)), ((), ())),
                                    preferred_element_type=jnp.float32)
            g[...] = h * dis[0][:, None]

        ab = a8v[pl.ds(i * BI, BI), :]
        gb = g[pl.ds(i * BI, BI), :].astype(jnp.bfloat16)
        part = jax.lax.dot_general(ab, gb, (((0,), (0,)), ((), ())),
                                   preferred_element_type=jnp.float32)

        @pl.when(i == 0)
        def _():
            acc[...] = part

        @pl.when(i > 0)
        def _():
            acc[...] += part

        @pl.when(i == NI - 1)
        def _():
            d = dis[0]
            h1c = jnp.maximum((acc[...] + g[...]) * d[:, None]
                              + b1_ref[...], 0.0)
            h2 = jax.lax.dot_general(h1c, w2_ref[...], (((1,), (1,)), ((), ())),
                                     preferred_element_type=jnp.float32)
            g[...] = h2 * d[:, None]

    @pl.when(phase == 2)
    def _():
        ab = a8v[pl.ds(i * BI, BI), :]
        gb = g[pl.ds(i * BI, BI), :].astype(jnp.bfloat16)
        part = jax.lax.dot_general(ab, gb, (((0,), (0,)), ((), ())),
                                   preferred_element_type=jnp.float32)

        @pl.when(i == 0)
        def _():
            acc[...] = part

        @pl.when(i > 0)
        def _():
            acc[...] += part

        @pl.when(i == NI - 1)
        def _():
            d = dis[0]
            h2c = jnp.maximum((acc[...] + g[...]) * d[:, None]
                              + b2_ref[...], 0.0)
            of = jax.lax.dot_general(h2c, wfc_ref[...], (((1,), (1,)), ((), ())),
                                     preferred_element_type=jnp.float32)
            of = of + bfc_ref[...]
            out_ref[0] = jnp.where(d[:, None] > 0, of, 0.0)


def kernel(big_batch_positions, big_batched_adjacency_pruned, ego_mask_batch,
           W1, b1, W2, b2, Wfc, bfc):
    x = big_batch_positions
    A = big_batched_adjacency_pruned
    m = jnp.transpose(ego_mask_batch, (1, 0, 2)).reshape(T, 1, BN)
    m = m.astype(jnp.float32)
    b1r = b1.reshape(1, HID)
    b2r = b2.reshape(1, HID)
    bfcr = bfc.reshape(1, OUT)

    out = pl.pallas_call(
        _fused_kernel, grid=(T, 3 * NI),
        in_specs=[
            pl.BlockSpec((1, 1, BN), lambda t, j: (t, 0, 0)),
            pl.BlockSpec((1, BI, BN),
                         lambda t, j: (t, jnp.minimum(j, NI - 1), 0)),
            pl.BlockSpec((1, BN, IN_DIM), lambda t, j: (t, 0, 0)),
            pl.BlockSpec((HID, IN_DIM), lambda t, j: (0, 0)),
            pl.BlockSpec((1, HID), lambda t, j: (0, 0)),
            pl.BlockSpec((HID, HID), lambda t, j: (0, 0)),
            pl.BlockSpec((1, HID), lambda t, j: (0, 0)),
            pl.BlockSpec((OUT, HID), lambda t, j: (0, 0)),
            pl.BlockSpec((1, OUT), lambda t, j: (0, 0)),
        ],
        out_specs=pl.BlockSpec((1, BN, OUT), lambda t, j: (t, 0, 0)),
        out_shape=jax.ShapeDtypeStruct((T, BN, OUT), jnp.float32),
        scratch_shapes=[
            pltpu.VMEM((BN, BN), jnp.bfloat16),
            pltpu.VMEM((BN, HID), jnp.float32),
            pltpu.VMEM((BN, HID), jnp.float32),
            pltpu.VMEM((1, BN), jnp.float32),
            pltpu.VMEM((1, BN), jnp.float32),
        ],
        compiler_params=pltpu.CompilerParams(
            dimension_semantics=("arbitrary", "arbitrary"),
            fuse_transposed_lhs_in_matmul=True,
            vmem_limit_bytes=100 * 1024 * 1024),
    )(m, A, x, W1, b1r, W2, b2r, Wfc, bfcr)

    h_stack = out.reshape(T, B, N, OUT)
    return jnp.transpose(h_stack, (1, 2, 0, 3))


# speedup vs baseline: 1.2786x; 1.2786x over previous
import jax
import jax.numpy as jnp
from jax.experimental import pallas as pl
from jax.experimental.pallas import tpu as pltpu

T, B, N = 4, 8, 512
BN = B * N
IN_DIM, HID, OUT = 128, 128, 64

BI = 512
NI = BN // BI


def _fused_kernel(m_ref, a_ref, x_ref, w1_ref, b1_ref, w2_ref, b2_ref,
                  wfc_ref, bfc_ref, out_ref, a8t, g1b, g2b, deg, dis):
    j = pl.program_id(1)
    phase = j // NI
    i = j % NI

    @pl.when(phase == 0)
    def _():
        a = a_ref[0]
        a8t[:, pl.ds(i * BI, BI)] = a.astype(jnp.bfloat16).T
        mi = m_ref[0, 0, pl.ds(i * BI, BI)]
        part = jnp.dot(mi[None, :], a, preferred_element_type=jnp.float32)

        @pl.when(i == 0)
        def _():
            deg[...] = part

        @pl.when(i > 0)
        def _():
            deg[...] += part

    @pl.when(phase == 1)
    def _():
        @pl.when(i == 0)
        def _():
            m = m_ref[0, 0]
            d = deg[0] * m + m
            dis[...] = jnp.where(d > 0, jax.lax.rsqrt(d), 0.0)[None]
            h = jax.lax.dot_general(x_ref[0], w1_ref[...],
                                    (((1,), (1,)), ((), ())),
                                    preferred_element_type=jnp.float32)
            g1b[...] = (h * dis[0][:, None]).astype(jnp.bfloat16)

        part = jax.lax.dot_general(a8t[pl.ds(i * BI, BI), :], g1b[...],
                                   (((1,), (0,)), ((), ())),
                                   preferred_element_type=jnp.float32)
        db = dis[0, pl.ds(i * BI, BI)]
        gj = g1b[pl.ds(i * BI, BI), :].astype(jnp.float32)
        h1c = jnp.maximum((part + gj) * db[:, None] + b1_ref[...], 0.0)
        h2 = jax.lax.dot_general(h1c, w2_ref[...], (((1,), (1,)), ((), ())),
                                 preferred_element_type=jnp.float32)
        g2b[pl.ds(i * BI, BI), :] = (h2 * db[:, None]).astype(jnp.bfloat16)

    @pl.when(phase == 2)
    def _():
        part = jax.lax.dot_general(a8t[pl.ds(i * BI, BI), :], g2b[...],
                                   (((1,), (0,)), ((), ())),
                                   preferred_element_type=jnp.float32)
        db = dis[0, pl.ds(i * BI, BI)]
        gj = g2b[pl.ds(i * BI, BI), :].astype(jnp.float32)
        h2c = jnp.maximum((part + gj) * db[:, None] + b2_ref[...], 0.0)
        of = jax.lax.dot_general(h2c, wfc_ref[...], (((1,), (1,)), ((), ())),
                                 preferred_element_type=jnp.float32)
        of = of + bfc_ref[...]
        out_ref[0] = jnp.where(db[:, None] > 0, of, 0.0)


def kernel(big_batch_positions, big_batched_adjacency_pruned, ego_mask_batch,
           W1, b1, W2, b2, Wfc, bfc):
    x = big_batch_positions
    A = big_batched_adjacency_pruned
    m = jnp.transpose(ego_mask_batch, (1, 0, 2)).reshape(T, 1, BN)
    m = m.astype(jnp.float32)
    b1r = b1.reshape(1, HID)
    b2r = b2.reshape(1, HID)
    bfcr = bfc.reshape(1, OUT)

    out = pl.pallas_call(
        _fused_kernel, grid=(T, 3 * NI),
        in_specs=[
            pl.BlockSpec((1, 1, BN), lambda t, j: (t, 0, 0)),
            pl.BlockSpec((1, BI, BN),
                         lambda t, j: (t, jnp.minimum(j, NI - 1), 0)),
            pl.BlockSpec((1, BN, IN_DIM), lambda t, j: (t, 0, 0)),
            pl.BlockSpec((HID, IN_DIM), lambda t, j: (0, 0)),
            pl.BlockSpec((1, HID), lambda t, j: (0, 0)),
            pl.BlockSpec((HID, HID), lambda t, j: (0, 0)),
            pl.BlockSpec((1, HID), lambda t, j: (0, 0)),
            pl.BlockSpec((OUT, HID), lambda t, j: (0, 0)),
            pl.BlockSpec((1, OUT), lambda t, j: (0, 0)),
        ],
        out_specs=pl.BlockSpec(
            (1, BI, OUT),
            lambda t, j: (t, jnp.clip(j - 2 * NI, 0, NI - 1), 0)),
        out_shape=jax.ShapeDtypeStruct((T, BN, OUT), jnp.float32),
        scratch_shapes=[
            pltpu.VMEM((BN, BN), jnp.bfloat16),
            pltpu.VMEM((BN, HID), jnp.bfloat16),
            pltpu.VMEM((BN, HID), jnp.bfloat16),
            pltpu.VMEM((1, BN), jnp.float32),
            pltpu.VMEM((1, BN), jnp.float32),
        ],
        compiler_params=pltpu.CompilerParams(
            dimension_semantics=("arbitrary", "arbitrary"),
            vmem_limit_bytes=100 * 1024 * 1024),
    )(m, A, x, W1, b1r, W2, b2r, Wfc, bfcr)

    h_stack = out.reshape(T, B, N, OUT)
    return jnp.transpose(h_stack, (1, 2, 0, 3))
